# Initial kernel scaffold; baseline (speedup 1.0000x reference)
import jax
import jax.numpy as jnp
from jax.experimental import pallas as pl


def kernel(state, action, edge_index, W1, b1, W2, b2, Wm1, bm1, Wm2, bm2, Wout, bout):
    raise NotImplementedError("write your pallas kernel here")


# speedup vs baseline: 31.5069x; 31.5069x over previous
import functools

import jax
import jax.numpy as jnp
from jax import lax
from jax.experimental import pallas as pl
from jax.experimental.pallas import tpu as pltpu
from jax.experimental.pallas import tpu_sc as plsc

N = 10000
NP = 10240
E = 320000
F = 16
NW = 32
CH = 128
CPT = 79
EP = NW * CPT * CH
RPT = NP // 16
BLK = 1024
GRID = NP // BLK

_mesh = plsc.VectorSubcoreMesh(core_axis_name="c", subcore_axis_name="s")


@functools.partial(
    pl.kernel,
    mesh=_mesh,
    out_type=jax.ShapeDtypeStruct((2, NP, F), jnp.float32),
    scratch_types=[
        pltpu.VMEM((CPT, CH), jnp.int32),
        pltpu.VMEM((CH, F), jnp.float32),
        pltpu.VMEM_SHARED((NP, F), jnp.float32),
    ],
    compiler_params=pltpu.CompilerParams(use_tc_tiling_on_sc=False),
)
def _deg_kernel(dst_hbm, ones_hbm, zeros_hbm, out_hbm, dst_v, ones_v, acc):
    c = lax.axis_index("c")
    s = lax.axis_index("s")
    wid = s * 2 + c
    pltpu.sync_copy(zeros_hbm, acc.at[pl.ds(s * RPT, RPT)])
    pltpu.sync_copy(dst_hbm.at[wid], dst_v)
    pltpu.sync_copy(ones_hbm, ones_v)
    plsc.subcore_barrier()

    def body(j, carry):
        pltpu.sync_copy(ones_v, acc.at[dst_v.at[j]], add=True)
        return carry

    lax.fori_loop(0, CPT, body, 0)
    plsc.subcore_barrier()
    pltpu.sync_copy(acc.at[pl.ds(s * RPT, RPT)],
                    out_hbm.at[c, pl.ds(s * RPT, RPT)])


@functools.partial(
    pl.kernel,
    mesh=_mesh,
    out_type=jax.ShapeDtypeStruct((2, NP, F), jnp.float32),
    scratch_types=[
        pltpu.VMEM((CPT, CH), jnp.int32),
        pltpu.VMEM((CPT, CH), jnp.int32),
        pltpu.VMEM((CH, F), jnp.float32),
        pltpu.VMEM_SHARED((NP, F), jnp.float32),
        pltpu.SemaphoreType.DMA,
    ],
    compiler_params=pltpu.CompilerParams(use_tc_tiling_on_sc=False),
)
def _agg_kernel(g_hbm, src_hbm, dst_hbm, zeros_hbm, out_hbm,
                src_v, dst_v, rows_v, acc, sem):
    c = lax.axis_index("c")
    s = lax.axis_index("s")
    wid = s * 2 + c
    pltpu.sync_copy(zeros_hbm, acc.at[pl.ds(s * RPT, RPT)])
    pltpu.sync_copy(src_hbm.at[wid], src_v)
    pltpu.sync_copy(dst_hbm.at[wid], dst_v)
    plsc.subcore_barrier()

    def body(j, carry):
        pltpu.async_copy(g_hbm.at[src_v.at[j]], rows_v, sem).wait()
        pltpu.sync_copy(rows_v, acc.at[dst_v.at[j]], add=True)
        return carry

    lax.fori_loop(0, CPT, body, 0)
    plsc.subcore_barrier()
    pltpu.sync_copy(acc.at[pl.ds(s * RPT, RPT)],
                    out_hbm.at[c, pl.ds(s * RPT, RPT)])


def _tc_a_body(state_ref, w1_ref, degrep_ref, h1_ref, g1_ref, dinv_ref):
    i = pl.program_id(0)
    h1 = jnp.dot(state_ref[...], w1_ref[...], preferred_element_type=jnp.float32)
    deg = degrep_ref[0] + degrep_ref[1] + 1.0
    rows = lax.broadcasted_iota(jnp.int32, (BLK, F), 0) + i * BLK
    dinv = jnp.where(rows < N, lax.rsqrt(deg), 0.0)
    h1_ref[...] = h1
    dinv_ref[...] = dinv
    g1_ref[...] = h1 * dinv


_tc_a = pl.pallas_call(
    _tc_a_body,
    grid=(GRID,),
    in_specs=[
        pl.BlockSpec((BLK, 128), lambda i: (i, 0)),
        pl.BlockSpec((128, F), lambda i: (0, 0)),
        pl.BlockSpec((2, BLK, F), lambda i: (0, i, 0)),
    ],
    out_specs=[
        pl.BlockSpec((BLK, F), lambda i: (i, 0)),
        pl.BlockSpec((BLK, F), lambda i: (i, 0)),
        pl.BlockSpec((BLK, F), lambda i: (i, 0)),
    ],
    out_shape=[
        jax.ShapeDtypeStruct((NP, F), jnp.float32),
        jax.ShapeDtypeStruct((NP, F), jnp.float32),
        jax.ShapeDtypeStruct((NP, F), jnp.float32),
    ],
)


def _tc_f_body(agg_ref, h1_ref, dinv_ref, b1_ref, x1_ref, g2_ref):
    dinv = dinv_ref[...]
    x1 = dinv * (agg_ref[0] + agg_ref[1]) + dinv * dinv * h1_ref[...] + b1_ref[...]
    x1 = jnp.maximum(x1, 0.0)
    x1_ref[...] = x1
    g2_ref[...] = x1 * dinv


_tc_f = pl.pallas_call(
    _tc_f_body,
    grid=(GRID,),
    in_specs=[
        pl.BlockSpec((2, BLK, F), lambda i: (0, i, 0)),
        pl.BlockSpec((BLK, F), lambda i: (i, 0)),
        pl.BlockSpec((BLK, F), lambda i: (i, 0)),
        pl.BlockSpec((1, F), lambda i: (0, 0)),
    ],
    out_specs=[
        pl.BlockSpec((BLK, F), lambda i: (i, 0)),
        pl.BlockSpec((BLK, F), lambda i: (i, 0)),
    ],
    out_shape=[
        jax.ShapeDtypeStruct((NP, F), jnp.float32),
        jax.ShapeDtypeStruct((NP, F), jnp.float32),
    ],
)


def _tc_g_body(agg_ref, x1_ref, dinv_ref, act_ref, w2_ref, b2_ref,
               wm1_ref, bm1_ref, wm2_ref, bm2_ref, woutT_ref, bout_ref, y_ref):
    dinv = dinv_ref[...]
    a2 = dinv * (agg_ref[0] + agg_ref[1]) + dinv * dinv * x1_ref[...]
    x2 = jnp.dot(a2, w2_ref[...], preferred_element_type=jnp.float32) + b2_ref[...]
    m1 = (jnp.dot(x2, wm1_ref[0:128], preferred_element_type=jnp.float32)
          + jnp.dot(act_ref[...], wm1_ref[128:192], preferred_element_type=jnp.float32)
          + bm1_ref[...])
    m1 = jnp.maximum(m1, 0.0)
    m2 = jnp.dot(m1, wm2_ref[...], preferred_element_type=jnp.float32) + bm2_ref[...]
    m2 = jnp.maximum(m2, 0.0)
    y = jnp.sum(m2 * woutT_ref[...], axis=1, keepdims=True) + bout_ref[0, 0]
    y_ref[...] = y


_tc_g = pl.pallas_call(
    _tc_g_body,
    grid=(GRID,),
    in_specs=[
        pl.BlockSpec((2, BLK, F), lambda i: (0, i, 0)),
        pl.BlockSpec((BLK, F), lambda i: (i, 0)),
        pl.BlockSpec((BLK, F), lambda i: (i, 0)),
        pl.BlockSpec((BLK, 64), lambda i: (i, 0)),
        pl.BlockSpec((F, 128), lambda i: (0, 0)),
        pl.BlockSpec((1, 128), lambda i: (0, 0)),
        pl.BlockSpec((192, 256), lambda i: (0, 0)),
        pl.BlockSpec((1, 256), lambda i: (0, 0)),
        pl.BlockSpec((256, 256), lambda i: (0, 0)),
        pl.BlockSpec((1, 256), lambda i: (0, 0)),
        pl.BlockSpec((1, 256), lambda i: (0, 0)),
        pl.BlockSpec((1, 128), lambda i: (0, 0)),
    ],
    out_specs=[pl.BlockSpec((BLK, 1), lambda i: (i, 0))],
    out_shape=[jax.ShapeDtypeStruct((NP, 1), jnp.float32)],
)


def kernel(state, action, edge_index, W1, b1, W2, b2,
           Wm1, bm1, Wm2, bm2, Wout, bout):
    f32 = jnp.float32
    state_p = jnp.zeros((NP, 128), f32).at[:N].set(state.astype(f32))
    act_p = jnp.zeros((NP, 64), f32).at[:N].set(action.astype(f32))

    ei = edge_index.astype(jnp.int32)
    src = jnp.full((EP,), NP - 1, jnp.int32).at[:E].set(ei[:, 0]).reshape(NW, CPT, CH)
    dst = jnp.full((EP,), N, jnp.int32).at[:E].set(ei[:, 1]).reshape(NW, CPT, CH)

    ones_rows = jnp.ones((CH, F), f32)
    zeros_rows = jnp.zeros((RPT, F), f32)

    degrep = _deg_kernel(dst, ones_rows, zeros_rows)
    h1, g1, dinv = _tc_a(state_p, W1.astype(f32), degrep)
    agg1 = _agg_kernel(g1, src, dst, zeros_rows)
    x1, g2 = _tc_f(agg1, h1, dinv, b1.reshape(1, F).astype(f32))
    agg2 = _agg_kernel(g2, src, dst, zeros_rows)
    (y,) = _tc_g(agg2, x1, dinv, act_p, W2.astype(f32),
                 b2.reshape(1, 128).astype(f32), Wm1.astype(f32),
                 bm1.reshape(1, 256).astype(f32), Wm2.astype(f32),
                 bm2.reshape(1, 256).astype(f32),
                 Wout.reshape(1, 256).astype(f32),
                 jnp.broadcast_to(bout.reshape(1, 1).astype(f32), (1, 128)))
    return y[:N]


# speedup vs baseline: 33.7630x; 1.0716x over previous
import functools

import jax
import jax.numpy as jnp
from jax import lax
from jax.experimental import pallas as pl
from jax.experimental.pallas import tpu as pltpu
from jax.experimental.pallas import tpu_sc as plsc

N = 10000
NP = 10240
E = 320000
F = 16
NW = 32
CH = 128
CPT = 80
KG = 4
NG = CPT // KG
EP = NW * CPT * CH
RPT = NP // 16
BLK = 1024
GRID = NP // BLK

_mesh = plsc.VectorSubcoreMesh(core_axis_name="c", subcore_axis_name="s")


@functools.partial(
    pl.kernel,
    mesh=_mesh,
    out_type=jax.ShapeDtypeStruct((2, NP, F), jnp.float32),
    scratch_types=[
        pltpu.VMEM((CPT, CH), jnp.int32),
        pltpu.VMEM((CH, F), jnp.float32),
        pltpu.VMEM_SHARED((NP, F), jnp.float32),
    ],
    compiler_params=pltpu.CompilerParams(use_tc_tiling_on_sc=False),
)
def _deg_kernel(dst_hbm, ones_hbm, zeros_hbm, out_hbm, dst_v, ones_v, acc):
    c = lax.axis_index("c")
    s = lax.axis_index("s")
    wid = s * 2 + c
    pltpu.sync_copy(zeros_hbm, acc.at[pl.ds(s * RPT, RPT)])
    pltpu.sync_copy(dst_hbm.at[wid], dst_v)
    pltpu.sync_copy(ones_hbm, ones_v)
    plsc.subcore_barrier()

    def body(j, carry):
        pltpu.sync_copy(ones_v, acc.at[dst_v.at[j]], add=True)
        return carry

    lax.fori_loop(0, CPT, body, 0)
    plsc.subcore_barrier()
    pltpu.sync_copy(acc.at[pl.ds(s * RPT, RPT)],
                    out_hbm.at[c, pl.ds(s * RPT, RPT)])


@functools.partial(
    pl.kernel,
    mesh=_mesh,
    out_type=jax.ShapeDtypeStruct((2, NP, F), jnp.float32),
    scratch_types=[
        pltpu.VMEM((CPT, CH), jnp.int32),
        pltpu.VMEM((CPT, CH), jnp.int32),
        pltpu.VMEM((2, KG, CH, F), jnp.float32),
        pltpu.VMEM_SHARED((NP, F), jnp.float32),
        pltpu.SemaphoreType.DMA,
        pltpu.SemaphoreType.DMA,
    ],
    compiler_params=pltpu.CompilerParams(use_tc_tiling_on_sc=False),
)
def _agg_kernel(g_hbm, src_hbm, dst_hbm, zeros_hbm, out_hbm,
                src_v, dst_v, bufs, acc, sem_a, sem_b):
    c = lax.axis_index("c")
    s = lax.axis_index("s")
    wid = s * 2 + c
    pltpu.sync_copy(zeros_hbm, acc.at[pl.ds(s * RPT, RPT)])
    pltpu.sync_copy(src_hbm.at[wid], src_v)
    pltpu.sync_copy(dst_hbm.at[wid], dst_v)
    plsc.subcore_barrier()

    sems = (sem_a, sem_b)

    def fire(group, half):
        for b in range(KG):
            pltpu.async_copy(g_hbm.at[src_v.at[group * KG + b]],
                             bufs.at[half, b], sems[half])

    def drain(half):
        for b in range(KG):
            pltpu.make_async_copy(g_hbm.at[pl.ds(0, CH)],
                                  bufs.at[half, b], sems[half]).wait()

    def scatter(group, half):
        for b in range(KG):
            pltpu.sync_copy(bufs.at[half, b],
                            acc.at[dst_v.at[group * KG + b]], add=True)

    fire(0, 0)

    def body(t, carry):
        g_a = 2 * t
        g_b = 2 * t + 1
        drain(0)
        fire(g_b, 1)
        scatter(g_a, 0)
        drain(1)
        fire(jnp.minimum(g_a + 2, NG - 1), 0)
        scatter(g_b, 1)
        return carry

    lax.fori_loop(0, NG // 2, body, 0)
    drain(0)
    plsc.subcore_barrier()
    pltpu.sync_copy(acc.at[pl.ds(s * RPT, RPT)],
                    out_hbm.at[c, pl.ds(s * RPT, RPT)])


def _tc_a_body(state_ref, w1_ref, degrep_ref, h1_ref, g1_ref, dinv_ref):
    i = pl.program_id(0)
    h1 = jnp.dot(state_ref[...], w1_ref[...], preferred_element_type=jnp.float32)
    deg = degrep_ref[0] + degrep_ref[1] + 1.0
    rows = lax.broadcasted_iota(jnp.int32, (BLK, F), 0) + i * BLK
    dinv = jnp.where(rows < N, lax.rsqrt(deg), 0.0)
    h1_ref[...] = h1
    dinv_ref[...] = dinv
    g1_ref[...] = h1 * dinv


_tc_a = pl.pallas_call(
    _tc_a_body,
    grid=(GRID,),
    in_specs=[
        pl.BlockSpec((BLK, 128), lambda i: (i, 0)),
        pl.BlockSpec((128, F), lambda i: (0, 0)),
        pl.BlockSpec((2, BLK, F), lambda i: (0, i, 0)),
    ],
    out_specs=[
        pl.BlockSpec((BLK, F), lambda i: (i, 0)),
        pl.BlockSpec((BLK, F), lambda i: (i, 0)),
        pl.BlockSpec((BLK, F), lambda i: (i, 0)),
    ],
    out_shape=[
        jax.ShapeDtypeStruct((NP, F), jnp.float32),
        jax.ShapeDtypeStruct((NP, F), jnp.float32),
        jax.ShapeDtypeStruct((NP, F), jnp.float32),
    ],
)


def _tc_f_body(agg_ref, h1_ref, dinv_ref, b1_ref, x1_ref, g2_ref):
    dinv = dinv_ref[...]
    x1 = dinv * (agg_ref[0] + agg_ref[1]) + dinv * dinv * h1_ref[...] + b1_ref[...]
    x1 = jnp.maximum(x1, 0.0)
    x1_ref[...] = x1
    g2_ref[...] = x1 * dinv


_tc_f = pl.pallas_call(
    _tc_f_body,
    grid=(GRID,),
    in_specs=[
        pl.BlockSpec((2, BLK, F), lambda i: (0, i, 0)),
        pl.BlockSpec((BLK, F), lambda i: (i, 0)),
        pl.BlockSpec((BLK, F), lambda i: (i, 0)),
        pl.BlockSpec((1, F), lambda i: (0, 0)),
    ],
    out_specs=[
        pl.BlockSpec((BLK, F), lambda i: (i, 0)),
        pl.BlockSpec((BLK, F), lambda i: (i, 0)),
    ],
    out_shape=[
        jax.ShapeDtypeStruct((NP, F), jnp.float32),
        jax.ShapeDtypeStruct((NP, F), jnp.float32),
    ],
)


def _tc_g_body(agg_ref, x1_ref, dinv_ref, act_ref, w2_ref, b2_ref,
               wm1_ref, bm1_ref, wm2_ref, bm2_ref, woutT_ref, bout_ref, y_ref):
    dinv = dinv_ref[...]
    a2 = dinv * (agg_ref[0] + agg_ref[1]) + dinv * dinv * x1_ref[...]
    x2 = jnp.dot(a2, w2_ref[...], preferred_element_type=jnp.float32) + b2_ref[...]
    m1 = (jnp.dot(x2, wm1_ref[0:128], preferred_element_type=jnp.float32)
          + jnp.dot(act_ref[...], wm1_ref[128:192], preferred_element_type=jnp.float32)
          + bm1_ref[...])
    m1 = jnp.maximum(m1, 0.0)
    m2 = jnp.dot(m1, wm2_ref[...], preferred_element_type=jnp.float32) + bm2_ref[...]
    m2 = jnp.maximum(m2, 0.0)
    y = jnp.sum(m2 * woutT_ref[...], axis=1, keepdims=True) + bout_ref[0, 0]
    y_ref[...] = y


_tc_g = pl.pallas_call(
    _tc_g_body,
    grid=(GRID,),
    in_specs=[
        pl.BlockSpec((2, BLK, F), lambda i: (0, i, 0)),
        pl.BlockSpec((BLK, F), lambda i: (i, 0)),
        pl.BlockSpec((BLK, F), lambda i: (i, 0)),
        pl.BlockSpec((BLK, 64), lambda i: (i, 0)),
        pl.BlockSpec((F, 128), lambda i: (0, 0)),
        pl.BlockSpec((1, 128), lambda i: (0, 0)),
        pl.BlockSpec((192, 256), lambda i: (0, 0)),
        pl.BlockSpec((1, 256), lambda i: (0, 0)),
        pl.BlockSpec((256, 256), lambda i: (0, 0)),
        pl.BlockSpec((1, 256), lambda i: (0, 0)),
        pl.BlockSpec((1, 256), lambda i: (0, 0)),
        pl.BlockSpec((1, 128), lambda i: (0, 0)),
    ],
    out_specs=[pl.BlockSpec((BLK, 1), lambda i: (i, 0))],
    out_shape=[jax.ShapeDtypeStruct((NP, 1), jnp.float32)],
)


def kernel(state, action, edge_index, W1, b1, W2, b2,
           Wm1, bm1, Wm2, bm2, Wout, bout):
    f32 = jnp.float32
    state_p = jnp.zeros((NP, 128), f32).at[:N].set(state.astype(f32))
    act_p = jnp.zeros((NP, 64), f32).at[:N].set(action.astype(f32))

    ei = edge_index.astype(jnp.int32)
    src = jnp.full((EP,), NP - 1, jnp.int32).at[:E].set(ei[:, 0]).reshape(NW, CPT, CH)
    dst = jnp.full((EP,), N, jnp.int32).at[:E].set(ei[:, 1]).reshape(NW, CPT, CH)

    ones_rows = jnp.ones((CH, F), f32)
    zeros_rows = jnp.zeros((RPT, F), f32)

    degrep = _deg_kernel(dst, ones_rows, zeros_rows)
    h1, g1, dinv = _tc_a(state_p, W1.astype(f32), degrep)
    agg1 = _agg_kernel(g1, src, dst, zeros_rows)
    x1, g2 = _tc_f(agg1, h1, dinv, b1.reshape(1, F).astype(f32))
    agg2 = _agg_kernel(g2, src, dst, zeros_rows)
    (y,) = _tc_g(agg2, x1, dinv, act_p, W2.astype(f32),
                 b2.reshape(1, 128).astype(f32), Wm1.astype(f32),
                 bm1.reshape(1, 256).astype(f32), Wm2.astype(f32),
                 bm2.reshape(1, 256).astype(f32),
                 Wout.reshape(1, 256).astype(f32),
                 jnp.broadcast_to(bout.reshape(1, 1).astype(f32), (1, 128)))
    return y[:N]


# speedup vs baseline: 37.1372x; 1.0999x over previous
import functools

import jax
import jax.numpy as jnp
from jax import lax
from jax.experimental import pallas as pl
from jax.experimental.pallas import tpu as pltpu
from jax.experimental.pallas import tpu_sc as plsc

N = 10000
NP = 10240
E = 320000
F = 16
NW = 32
CH = 128
CPT = 80
KG = 8
NG = CPT // KG
EP = NW * CPT * CH
RPT = NP // 16
BLK = 1024
GRID = NP // BLK

_mesh = plsc.VectorSubcoreMesh(core_axis_name="c", subcore_axis_name="s")


@functools.partial(
    pl.kernel,
    mesh=_mesh,
    out_type=jax.ShapeDtypeStruct((2, NP, F), jnp.float32),
    scratch_types=[
        pltpu.VMEM((CPT, CH), jnp.int32),
        pltpu.VMEM((CH, F), jnp.float32),
        pltpu.VMEM_SHARED((NP, F), jnp.float32),
        pltpu.SemaphoreType.DMA,
    ],
    compiler_params=pltpu.CompilerParams(use_tc_tiling_on_sc=False),
)
def _deg_kernel(dst_hbm, ones_hbm, zeros_hbm, out_hbm, dst_v, ones_v, acc,
                sem):
    c = lax.axis_index("c")
    s = lax.axis_index("s")
    wid = s * 2 + c
    pltpu.sync_copy(zeros_hbm, acc.at[pl.ds(s * RPT, RPT)])
    pltpu.sync_copy(dst_hbm.at[wid], dst_v)
    pltpu.sync_copy(ones_hbm, ones_v)
    plsc.subcore_barrier()

    def body(t, carry):
        for b in range(KG):
            pltpu.async_copy(ones_v, acc.at[dst_v.at[t * KG + b]], sem,
                             add=True)
        for b in range(KG):
            pltpu.make_async_copy(ones_hbm, ones_v, sem).wait()
        return carry

    lax.fori_loop(0, CPT // KG, body, 0)
    plsc.subcore_barrier()
    pltpu.sync_copy(acc.at[pl.ds(s * RPT, RPT)],
                    out_hbm.at[c, pl.ds(s * RPT, RPT)])


@functools.partial(
    pl.kernel,
    mesh=_mesh,
    out_type=jax.ShapeDtypeStruct((2, NP, F), jnp.float32),
    scratch_types=[
        pltpu.VMEM((CPT, CH), jnp.int32),
        pltpu.VMEM((CPT, CH), jnp.int32),
        pltpu.VMEM((2, KG, CH, F), jnp.float32),
        pltpu.VMEM_SHARED((NP, F), jnp.float32),
        pltpu.SemaphoreType.DMA,
        pltpu.SemaphoreType.DMA,
    ],
    compiler_params=pltpu.CompilerParams(use_tc_tiling_on_sc=False),
)
def _agg_kernel(g_hbm, src_hbm, dst_hbm, zeros_hbm, out_hbm,
                src_v, dst_v, bufs, acc, sem_a, sem_b):
    c = lax.axis_index("c")
    s = lax.axis_index("s")
    wid = s * 2 + c
    pltpu.sync_copy(zeros_hbm, acc.at[pl.ds(s * RPT, RPT)])
    pltpu.sync_copy(src_hbm.at[wid], src_v)
    pltpu.sync_copy(dst_hbm.at[wid], dst_v)
    plsc.subcore_barrier()

    sems = (sem_a, sem_b)

    def fire(group, half):
        for b in range(KG):
            pltpu.async_copy(g_hbm.at[src_v.at[group * KG + b]],
                             bufs.at[half, b], sems[half])

    def drain(half):
        for b in range(KG):
            pltpu.make_async_copy(g_hbm.at[pl.ds(0, CH)],
                                  bufs.at[half, b], sems[half]).wait()

    def scatter(group, half):
        for b in range(KG):
            pltpu.sync_copy(bufs.at[half, b],
                            acc.at[dst_v.at[group * KG + b]], add=True)

    fire(0, 0)

    def body(t, carry):
        g_a = 2 * t
        g_b = 2 * t + 1
        drain(0)
        fire(g_b, 1)
        scatter(g_a, 0)
        drain(1)
        fire(jnp.minimum(g_a + 2, NG - 1), 0)
        scatter(g_b, 1)
        return carry

    lax.fori_loop(0, NG // 2, body, 0)
    drain(0)
    plsc.subcore_barrier()
    pltpu.sync_copy(acc.at[pl.ds(s * RPT, RPT)],
                    out_hbm.at[c, pl.ds(s * RPT, RPT)])


def _tc_a1_body(state_ref, w1_ref, h1_ref):
    h1_ref[...] = jnp.dot(state_ref[...], w1_ref[...],
                          preferred_element_type=jnp.float32)


_tc_a1 = pl.pallas_call(
    _tc_a1_body,
    grid=(GRID,),
    in_specs=[
        pl.BlockSpec((BLK, 128), lambda i: (i, 0)),
        pl.BlockSpec((128, F), lambda i: (0, 0)),
    ],
    out_specs=[pl.BlockSpec((BLK, F), lambda i: (i, 0))],
    out_shape=[jax.ShapeDtypeStruct((NP, F), jnp.float32)],
)


def _tc_a2_body(degrep_ref, h1_ref, g1_ref, dinv_ref):
    i = pl.program_id(0)
    deg = degrep_ref[0] + degrep_ref[1] + 1.0
    rows = lax.broadcasted_iota(jnp.int32, (BLK, F), 0) + i * BLK
    dinv = jnp.where(rows < N, lax.rsqrt(deg), 0.0)
    dinv_ref[...] = dinv
    g1_ref[...] = h1_ref[...] * dinv


_tc_a2 = pl.pallas_call(
    _tc_a2_body,
    grid=(GRID,),
    in_specs=[
        pl.BlockSpec((2, BLK, F), lambda i: (0, i, 0)),
        pl.BlockSpec((BLK, F), lambda i: (i, 0)),
    ],
    out_specs=[
        pl.BlockSpec((BLK, F), lambda i: (i, 0)),
        pl.BlockSpec((BLK, F), lambda i: (i, 0)),
    ],
    out_shape=[
        jax.ShapeDtypeStruct((NP, F), jnp.float32),
        jax.ShapeDtypeStruct((NP, F), jnp.float32),
    ],
)


def _tc_f_body(agg_ref, h1_ref, dinv_ref, b1_ref, x1_ref, g2_ref):
    dinv = dinv_ref[...]
    x1 = dinv * (agg_ref[0] + agg_ref[1]) + dinv * dinv * h1_ref[...] + b1_ref[...]
    x1 = jnp.maximum(x1, 0.0)
    x1_ref[...] = x1
    g2_ref[...] = x1 * dinv


_tc_f = pl.pallas_call(
    _tc_f_body,
    grid=(GRID,),
    in_specs=[
        pl.BlockSpec((2, BLK, F), lambda i: (0, i, 0)),
        pl.BlockSpec((BLK, F), lambda i: (i, 0)),
        pl.BlockSpec((BLK, F), lambda i: (i, 0)),
        pl.BlockSpec((1, F), lambda i: (0, 0)),
    ],
    out_specs=[
        pl.BlockSpec((BLK, F), lambda i: (i, 0)),
        pl.BlockSpec((BLK, F), lambda i: (i, 0)),
    ],
    out_shape=[
        jax.ShapeDtypeStruct((NP, F), jnp.float32),
        jax.ShapeDtypeStruct((NP, F), jnp.float32),
    ],
)


def _tc_g_body(agg_ref, x1_ref, dinv_ref, act_ref, w2_ref, b2_ref,
               wm1_ref, bm1_ref, wm2_ref, bm2_ref, woutT_ref, bout_ref, y_ref):
    dinv = dinv_ref[...]
    a2 = dinv * (agg_ref[0] + agg_ref[1]) + dinv * dinv * x1_ref[...]
    x2 = jnp.dot(a2, w2_ref[...], preferred_element_type=jnp.float32) + b2_ref[...]
    m1 = (jnp.dot(x2, wm1_ref[0:128], preferred_element_type=jnp.float32)
          + jnp.dot(act_ref[...], wm1_ref[128:192], preferred_element_type=jnp.float32)
          + bm1_ref[...])
    m1 = jnp.maximum(m1, 0.0)
    m2 = jnp.dot(m1, wm2_ref[...], preferred_element_type=jnp.float32) + bm2_ref[...]
    m2 = jnp.maximum(m2, 0.0)
    y = jnp.sum(m2 * woutT_ref[...], axis=1, keepdims=True) + bout_ref[0, 0]
    y_ref[...] = y


_tc_g = pl.pallas_call(
    _tc_g_body,
    grid=(GRID,),
    in_specs=[
        pl.BlockSpec((2, BLK, F), lambda i: (0, i, 0)),
        pl.BlockSpec((BLK, F), lambda i: (i, 0)),
        pl.BlockSpec((BLK, F), lambda i: (i, 0)),
        pl.BlockSpec((BLK, 64), lambda i: (i, 0)),
        pl.BlockSpec((F, 128), lambda i: (0, 0)),
        pl.BlockSpec((1, 128), lambda i: (0, 0)),
        pl.BlockSpec((192, 256), lambda i: (0, 0)),
        pl.BlockSpec((1, 256), lambda i: (0, 0)),
        pl.BlockSpec((256, 256), lambda i: (0, 0)),
        pl.BlockSpec((1, 256), lambda i: (0, 0)),
        pl.BlockSpec((1, 256), lambda i: (0, 0)),
        pl.BlockSpec((1, 128), lambda i: (0, 0)),
    ],
    out_specs=[pl.BlockSpec((BLK, 1), lambda i: (i, 0))],
    out_shape=[jax.ShapeDtypeStruct((NP, 1), jnp.float32)],
)


def kernel(state, action, edge_index, W1, b1, W2, b2,
           Wm1, bm1, Wm2, bm2, Wout, bout):
    f32 = jnp.float32
    state_p = jnp.zeros((NP, 128), f32).at[:N].set(state.astype(f32))
    act_p = jnp.zeros((NP, 64), f32).at[:N].set(action.astype(f32))

    ei = edge_index.astype(jnp.int32)
    src = jnp.full((EP,), NP - 1, jnp.int32).at[:E].set(ei[:, 0]).reshape(NW, CPT, CH)
    dst = jnp.full((EP,), N, jnp.int32).at[:E].set(ei[:, 1]).reshape(NW, CPT, CH)

    ones_rows = jnp.ones((CH, F), f32)
    zeros_rows = jnp.zeros((RPT, F), f32)

    degrep = _deg_kernel(dst, ones_rows, zeros_rows)
    (h1,) = _tc_a1(state_p, W1.astype(f32))
    g1, dinv = _tc_a2(degrep, h1)
    agg1 = _agg_kernel(g1, src, dst, zeros_rows)
    x1, g2 = _tc_f(agg1, h1, dinv, b1.reshape(1, F).astype(f32))
    agg2 = _agg_kernel(g2, src, dst, zeros_rows)
    (y,) = _tc_g(agg2, x1, dinv, act_p, W2.astype(f32),
                 b2.reshape(1, 128).astype(f32), Wm1.astype(f32),
                 bm1.reshape(1, 256).astype(f32), Wm2.astype(f32),
                 bm2.reshape(1, 256).astype(f32),
                 Wout.reshape(1, 256).astype(f32),
                 jnp.broadcast_to(bout.reshape(1, 1).astype(f32), (1, 128)))
    return y[:N]


# speedup vs baseline: 37.1395x; 1.0001x over previous
import functools

import jax
import jax.numpy as jnp
from jax import lax
from jax.experimental import pallas as pl
from jax.experimental.pallas import tpu as pltpu
from jax.experimental.pallas import tpu_sc as plsc

N = 10000
NP = 10240
E = 320000
F = 16
NW = 32
CH = 128
KG = 8
CPT0 = 112
CPT1 = 48
NCHUNK = 16 * (CPT0 + CPT1)
NCHUNK_PAD = 16 * CPT0 + 15 * CPT1 + CPT0
EP = NCHUNK_PAD * CH
RPT = NP // 16
BLK = 1024
GRID = NP // BLK

_mesh = plsc.VectorSubcoreMesh(core_axis_name="c", subcore_axis_name="s")


@functools.partial(
    pl.kernel,
    mesh=_mesh,
    out_type=jax.ShapeDtypeStruct((2, NP, F), jnp.float32),
    scratch_types=[
        pltpu.VMEM((CPT0, CH), jnp.int32),
        pltpu.VMEM((CH, F), jnp.float32),
        pltpu.VMEM_SHARED((NP, F), jnp.float32),
        pltpu.SemaphoreType.DMA,
    ],
    compiler_params=pltpu.CompilerParams(use_tc_tiling_on_sc=False),
)
def _deg_kernel(dst_hbm, ones_hbm, zeros_hbm, out_hbm, dst_v, ones_v, acc,
                sem):
    c = lax.axis_index("c")
    s = lax.axis_index("s")
    start = jnp.where(c == 0, s * CPT0, 16 * CPT0 + s * CPT1)
    ng = jnp.where(c == 0, CPT0 // KG, CPT1 // KG)
    pltpu.sync_copy(zeros_hbm, acc.at[pl.ds(s * RPT, RPT)])
    pltpu.sync_copy(dst_hbm.at[pl.ds(start, CPT0)], dst_v)
    pltpu.sync_copy(ones_hbm, ones_v)
    plsc.subcore_barrier()

    def body(t, carry):
        for b in range(KG):
            pltpu.async_copy(ones_v, acc.at[dst_v.at[t * KG + b]], sem,
                             add=True)
        for b in range(KG):
            pltpu.make_async_copy(ones_hbm, ones_v, sem).wait()
        return carry

    lax.fori_loop(0, ng, body, 0)
    plsc.subcore_barrier()
    pltpu.sync_copy(acc.at[pl.ds(s * RPT, RPT)],
                    out_hbm.at[c, pl.ds(s * RPT, RPT)])


@functools.partial(
    pl.kernel,
    mesh=_mesh,
    out_type=jax.ShapeDtypeStruct((2, NP, F), jnp.float32),
    scratch_types=[
        pltpu.VMEM((CPT0, CH), jnp.int32),
        pltpu.VMEM((CPT0, CH), jnp.int32),
        pltpu.VMEM((2, KG, CH, F), jnp.float32),
        pltpu.VMEM_SHARED((NP, F), jnp.float32),
        pltpu.SemaphoreType.DMA,
        pltpu.SemaphoreType.DMA,
    ],
    compiler_params=pltpu.CompilerParams(use_tc_tiling_on_sc=False),
)
def _agg_kernel(g_hbm, src_hbm, dst_hbm, zeros_hbm, out_hbm,
                src_v, dst_v, bufs, acc, sem_a, sem_b):
    c = lax.axis_index("c")
    s = lax.axis_index("s")
    start = jnp.where(c == 0, s * CPT0, 16 * CPT0 + s * CPT1)
    ng = jnp.where(c == 0, CPT0 // KG, CPT1 // KG)
    pltpu.sync_copy(zeros_hbm, acc.at[pl.ds(s * RPT, RPT)])
    pltpu.sync_copy(src_hbm.at[pl.ds(start, CPT0)], src_v)
    pltpu.sync_copy(dst_hbm.at[pl.ds(start, CPT0)], dst_v)
    plsc.subcore_barrier()

    sems = (sem_a, sem_b)

    def fire(group, half):
        for b in range(KG):
            pltpu.async_copy(g_hbm.at[src_v.at[group * KG + b]],
                             bufs.at[half, b], sems[half])

    def drain(half):
        for b in range(KG):
            pltpu.make_async_copy(g_hbm.at[pl.ds(0, CH)],
                                  bufs.at[half, b], sems[half]).wait()

    def scatter(group, half):
        for b in range(KG):
            pltpu.sync_copy(bufs.at[half, b],
                            acc.at[dst_v.at[group * KG + b]], add=True)

    fire(0, 0)

    def body(t, carry):
        g_a = 2 * t
        g_b = 2 * t + 1
        drain(0)
        fire(g_b, 1)
        scatter(g_a, 0)
        drain(1)
        fire(jnp.minimum(g_a + 2, ng - 1), 0)
        scatter(g_b, 1)
        return carry

    lax.fori_loop(0, ng // 2, body, 0)
    drain(0)
    plsc.subcore_barrier()
    pltpu.sync_copy(acc.at[pl.ds(s * RPT, RPT)],
                    out_hbm.at[c, pl.ds(s * RPT, RPT)])


def _tc_a1_body(state_ref, w1_ref, h1_ref):
    h1_ref[...] = jnp.dot(state_ref[...], w1_ref[...],
                          preferred_element_type=jnp.float32)


_tc_a1 = pl.pallas_call(
    _tc_a1_body,
    grid=(GRID,),
    in_specs=[
        pl.BlockSpec((BLK, 128), lambda i: (i, 0)),
        pl.BlockSpec((128, F), lambda i: (0, 0)),
    ],
    out_specs=[pl.BlockSpec((BLK, F), lambda i: (i, 0))],
    out_shape=[jax.ShapeDtypeStruct((NP, F), jnp.float32)],
)


def _tc_a2_body(degrep_ref, h1_ref, g1_ref, dinv_ref):
    i = pl.program_id(0)
    deg = degrep_ref[0] + degrep_ref[1] + 1.0
    rows = lax.broadcasted_iota(jnp.int32, (BLK, F), 0) + i * BLK
    dinv = jnp.where(rows < N, lax.rsqrt(deg), 0.0)
    dinv_ref[...] = dinv
    g1_ref[...] = h1_ref[...] * dinv


_tc_a2 = pl.pallas_call(
    _tc_a2_body,
    grid=(GRID,),
    in_specs=[
        pl.BlockSpec((2, BLK, F), lambda i: (0, i, 0)),
        pl.BlockSpec((BLK, F), lambda i: (i, 0)),
    ],
    out_specs=[
        pl.BlockSpec((BLK, F), lambda i: (i, 0)),
        pl.BlockSpec((BLK, F), lambda i: (i, 0)),
    ],
    out_shape=[
        jax.ShapeDtypeStruct((NP, F), jnp.float32),
        jax.ShapeDtypeStruct((NP, F), jnp.float32),
    ],
)


def _tc_f_body(agg_ref, h1_ref, dinv_ref, b1_ref, x1_ref, g2_ref):
    dinv = dinv_ref[...]
    x1 = dinv * (agg_ref[0] + agg_ref[1]) + dinv * dinv * h1_ref[...] + b1_ref[...]
    x1 = jnp.maximum(x1, 0.0)
    x1_ref[...] = x1
    g2_ref[...] = x1 * dinv


_tc_f = pl.pallas_call(
    _tc_f_body,
    grid=(GRID,),
    in_specs=[
        pl.BlockSpec((2, BLK, F), lambda i: (0, i, 0)),
        pl.BlockSpec((BLK, F), lambda i: (i, 0)),
        pl.BlockSpec((BLK, F), lambda i: (i, 0)),
        pl.BlockSpec((1, F), lambda i: (0, 0)),
    ],
    out_specs=[
        pl.BlockSpec((BLK, F), lambda i: (i, 0)),
        pl.BlockSpec((BLK, F), lambda i: (i, 0)),
    ],
    out_shape=[
        jax.ShapeDtypeStruct((NP, F), jnp.float32),
        jax.ShapeDtypeStruct((NP, F), jnp.float32),
    ],
)


def _tc_g_body(agg_ref, x1_ref, dinv_ref, act_ref, w2_ref, b2_ref,
               wm1_ref, bm1_ref, wm2_ref, bm2_ref, woutT_ref, bout_ref, y_ref):
    dinv = dinv_ref[...]
    a2 = dinv * (agg_ref[0] + agg_ref[1]) + dinv * dinv * x1_ref[...]
    x2 = jnp.dot(a2, w2_ref[...], preferred_element_type=jnp.float32) + b2_ref[...]
    m1 = (jnp.dot(x2, wm1_ref[0:128], preferred_element_type=jnp.float32)
          + jnp.dot(act_ref[...], wm1_ref[128:192], preferred_element_type=jnp.float32)
          + bm1_ref[...])
    m1 = jnp.maximum(m1, 0.0)
    m2 = jnp.dot(m1, wm2_ref[...], preferred_element_type=jnp.float32) + bm2_ref[...]
    m2 = jnp.maximum(m2, 0.0)
    y = jnp.sum(m2 * woutT_ref[...], axis=1, keepdims=True) + bout_ref[0, 0]
    y_ref[...] = y


_tc_g = pl.pallas_call(
    _tc_g_body,
    grid=(GRID,),
    in_specs=[
        pl.BlockSpec((2, BLK, F), lambda i: (0, i, 0)),
        pl.BlockSpec((BLK, F), lambda i: (i, 0)),
        pl.BlockSpec((BLK, F), lambda i: (i, 0)),
        pl.BlockSpec((BLK, 64), lambda i: (i, 0)),
        pl.BlockSpec((F, 128), lambda i: (0, 0)),
        pl.BlockSpec((1, 128), lambda i: (0, 0)),
        pl.BlockSpec((192, 256), lambda i: (0, 0)),
        pl.BlockSpec((1, 256), lambda i: (0, 0)),
        pl.BlockSpec((256, 256), lambda i: (0, 0)),
        pl.BlockSpec((1, 256), lambda i: (0, 0)),
        pl.BlockSpec((1, 256), lambda i: (0, 0)),
        pl.BlockSpec((1, 128), lambda i: (0, 0)),
    ],
    out_specs=[pl.BlockSpec((BLK, 1), lambda i: (i, 0))],
    out_shape=[jax.ShapeDtypeStruct((NP, 1), jnp.float32)],
)


def kernel(state, action, edge_index, W1, b1, W2, b2,
           Wm1, bm1, Wm2, bm2, Wout, bout):
    f32 = jnp.float32
    state_p = jnp.zeros((NP, 128), f32).at[:N].set(state.astype(f32))
    act_p = jnp.zeros((NP, 64), f32).at[:N].set(action.astype(f32))

    ei = edge_index.astype(jnp.int32)
    src = jnp.full((EP,), NP - 1, jnp.int32).at[:E].set(ei[:, 0]).reshape(NCHUNK_PAD, CH)
    dst = jnp.full((EP,), N, jnp.int32).at[:E].set(ei[:, 1]).reshape(NCHUNK_PAD, CH)

    ones_rows = jnp.ones((CH, F), f32)
    zeros_rows = jnp.zeros((RPT, F), f32)

    degrep = _deg_kernel(dst, ones_rows, zeros_rows)
    (h1,) = _tc_a1(state_p, W1.astype(f32))
    g1, dinv = _tc_a2(degrep, h1)
    agg1 = _agg_kernel(g1, src, dst, zeros_rows)
    x1, g2 = _tc_f(agg1, h1, dinv, b1.reshape(1, F).astype(f32))
    agg2 = _agg_kernel(g2, src, dst, zeros_rows)
    (y,) = _tc_g(agg2, x1, dinv, act_p, W2.astype(f32),
                 b2.reshape(1, 128).astype(f32), Wm1.astype(f32),
                 bm1.reshape(1, 256).astype(f32), Wm2.astype(f32),
                 bm2.reshape(1, 256).astype(f32),
                 Wout.reshape(1, 256).astype(f32),
                 jnp.broadcast_to(bout.reshape(1, 1).astype(f32), (1, 128)))
    return y[:N]
